# baseline (speedup 1.0000x reference)
import functools

import jax
import jax.numpy as jnp
from jax import lax
from jax.experimental import pallas as pl
from jax.experimental.pallas import tpu as pltpu
from jax.experimental.pallas import tpu_sc as plsc

H = 768
L = 16
NC = 2
NS = 16
NW = NC * NS
SEQ = 512
EPS = 1e-5


def _sc_gather_body(ids_hbm, we_hbm, out_hbm, idx_v, rows_v,
                    gsem0, gsem1, osem0, osem1, *, n_rows, k):
    cid = lax.axis_index("c")
    sid = lax.axis_index("s")
    wid = sid * NC + cid
    rows_per_worker = n_rows // NW
    n_chunks = rows_per_worker // k
    gsems = (gsem0, gsem1)
    osems = (osem0, osem1)
    base = wid * rows_per_worker

    def gather_start(g, b):
        row0 = base + g * k
        pltpu.sync_copy(ids_hbm.at[pl.ds(row0, k)], idx_v.at[b])
        pltpu.async_copy(we_hbm.at[idx_v.at[b]], rows_v.at[b], gsems[b])

    def gather_wait(b):
        pltpu.make_async_copy(we_hbm.at[idx_v.at[b]], rows_v.at[b],
                              gsems[b]).wait()

    def out_start(g, b):
        row0 = base + g * k
        pltpu.async_copy(rows_v.at[b], out_hbm.at[pl.ds(row0, k)], osems[b])

    def out_wait(b):
        pltpu.make_async_copy(rows_v.at[b], out_hbm.at[pl.ds(base, k)],
                              osems[b]).wait()

    gather_start(0, 0)

    def pipe(i, carry):
        for half in range(2):
            b = half
            nb = 1 - half
            g = 2 * i + half
            ng = g + 1

            @pl.when(ng < n_chunks)
            def _prefetch():
                @pl.when(ng >= 2)
                def _():
                    out_wait(nb)

                gather_start(ng, nb)

            gather_wait(b)
            out_start(g, b)
        return carry

    lax.fori_loop(0, n_chunks // 2, pipe, 0)
    out_wait(0)
    out_wait(1)


def _sc_gather(input_ids_flat, word_embeddings, n_rows):
    k = 64
    mesh = plsc.VectorSubcoreMesh(core_axis_name="c", subcore_axis_name="s",
                                  num_cores=NC, num_subcores=NS)
    body = functools.partial(_sc_gather_body, n_rows=n_rows, k=k)
    run = pl.kernel(
        body,
        out_type=jax.ShapeDtypeStruct((n_rows, H), jnp.float32),
        mesh=mesh,
        compiler_params=pltpu.CompilerParams(needs_layout_passes=False),
        scratch_types=[
            pltpu.VMEM((2, k), jnp.int32),
            pltpu.VMEM((2, k, H), jnp.float32),
            pltpu.SemaphoreType.DMA,
            pltpu.SemaphoreType.DMA,
            pltpu.SemaphoreType.DMA,
            pltpu.SemaphoreType.DMA,
        ],
        name="bert_embed_gather_sc",
    )
    return run(input_ids_flat, word_embeddings)


def _tc_ln_body(g_ref, pe_ref, tte_ref, gamma_ref, beta_ref, o_ref):
    x = g_ref[...] + pe_ref[...] + tte_ref[...]
    mean = jnp.mean(x, axis=1, keepdims=True)
    xc = x - mean
    var = jnp.mean(xc * xc, axis=1, keepdims=True)
    o_ref[...] = (xc * lax.rsqrt(var + EPS)) * gamma_ref[...] + beta_ref[...]


def _tc_ln(gathered, position_embeddings, tte_row, gamma2d, beta2d, n_rows):
    n_blocks = n_rows // SEQ
    return pl.pallas_call(
        _tc_ln_body,
        grid=(n_blocks,),
        in_specs=[
            pl.BlockSpec((SEQ, H), lambda i: (i, 0)),
            pl.BlockSpec((SEQ, H), lambda i: (0, 0)),
            pl.BlockSpec((1, H), lambda i: (0, 0)),
            pl.BlockSpec((1, H), lambda i: (0, 0)),
            pl.BlockSpec((1, H), lambda i: (0, 0)),
        ],
        out_specs=pl.BlockSpec((SEQ, H), lambda i: (i, 0)),
        out_shape=jax.ShapeDtypeStruct((n_rows, H), jnp.float32),
        name="bert_embed_ln_tc",
    )(gathered, position_embeddings, tte_row, gamma2d, beta2d)


def kernel(input_ids, word_embeddings, position_embeddings,
           token_type_embeddings, ln_gamma, ln_beta):
    b, seq = input_ids.shape
    n_rows = b * seq
    nparts = 4
    part = n_rows // nparts
    ids_flat = input_ids.reshape(n_rows)
    tte_row = token_type_embeddings[0].reshape(1, H)
    gamma2d = ln_gamma.reshape(1, H)
    beta2d = ln_beta.reshape(1, H)
    outs = []
    for p in range(nparts):
        gathered = _sc_gather(lax.dynamic_slice_in_dim(ids_flat, p * part, part),
                              word_embeddings, part)
        outs.append(_tc_ln(gathered, position_embeddings, tte_row,
                           gamma2d, beta2d, part))
    return jnp.concatenate(outs, axis=0).reshape(b, seq, H)


# speedup vs baseline: 1.4047x; 1.4047x over previous
import functools

import jax
import jax.numpy as jnp
from jax import lax
from jax.experimental import pallas as pl
from jax.experimental.pallas import tpu as pltpu
from jax.experimental.pallas import tpu_sc as plsc

H = 768
L = 16
NC = 2
NS = 16
NW = NC * NS
SEQ = 512
EPS = 1e-5


def _sc_gather_body(ids_hbm, we_hbm, out_hbm, idx_v, rows_v,
                    gsem0, gsem1, osem0, osem1, *, n_rows, k):
    cid = lax.axis_index("c")
    sid = lax.axis_index("s")
    wid = sid * NC + cid
    rows_per_worker = n_rows // NW
    n_chunks = rows_per_worker // k
    gsems = (gsem0, gsem1)
    osems = (osem0, osem1)
    base = wid * rows_per_worker

    def gather_start(g, b):
        row0 = base + g * k
        pltpu.sync_copy(ids_hbm.at[pl.ds(row0, k)], idx_v.at[b])
        pltpu.async_copy(we_hbm.at[idx_v.at[b]], rows_v.at[b], gsems[b])

    def gather_wait(b):
        pltpu.make_async_copy(we_hbm.at[idx_v.at[b]], rows_v.at[b],
                              gsems[b]).wait()

    def out_start(g, b):
        row0 = base + g * k
        pltpu.async_copy(rows_v.at[b], out_hbm.at[pl.ds(row0, k)], osems[b])

    def out_wait(b):
        pltpu.make_async_copy(rows_v.at[b], out_hbm.at[pl.ds(base, k)],
                              osems[b]).wait()

    gather_start(0, 0)

    def pipe(i, carry):
        for half in range(2):
            b = half
            nb = 1 - half
            g = 2 * i + half
            ng = g + 1

            @pl.when(ng < n_chunks)
            def _prefetch():
                @pl.when(ng >= 2)
                def _():
                    out_wait(nb)

                gather_start(ng, nb)

            gather_wait(b)
            out_start(g, b)
        return carry

    lax.fori_loop(0, n_chunks // 2, pipe, 0)
    out_wait(0)
    out_wait(1)


def _sc_gather(input_ids_flat, word_embeddings, n_rows):
    k = 64
    mesh = plsc.VectorSubcoreMesh(core_axis_name="c", subcore_axis_name="s",
                                  num_cores=NC, num_subcores=NS)
    body = functools.partial(_sc_gather_body, n_rows=n_rows, k=k)
    run = pl.kernel(
        body,
        out_type=jax.ShapeDtypeStruct((n_rows, H), jnp.float32),
        mesh=mesh,
        compiler_params=pltpu.CompilerParams(needs_layout_passes=False),
        scratch_types=[
            pltpu.VMEM((2, k), jnp.int32),
            pltpu.VMEM((2, k, H), jnp.float32),
            pltpu.SemaphoreType.DMA,
            pltpu.SemaphoreType.DMA,
            pltpu.SemaphoreType.DMA,
            pltpu.SemaphoreType.DMA,
        ],
        name="bert_embed_gather_sc",
    )
    return run(input_ids_flat, word_embeddings)


def _tc_ln_body(g_ref, pe_ref, tte_ref, gamma_ref, beta_ref, o_ref):
    x = g_ref[...] + pe_ref[...] + tte_ref[...]
    mean = jnp.mean(x, axis=1, keepdims=True)
    xc = x - mean
    var = jnp.mean(xc * xc, axis=1, keepdims=True)
    o_ref[...] = (xc * lax.rsqrt(var + EPS)) * gamma_ref[...] + beta_ref[...]


def _tc_ln(gathered, position_embeddings, tte_row, gamma2d, beta2d, n_rows):
    n_blocks = n_rows // SEQ
    return pl.pallas_call(
        _tc_ln_body,
        grid=(n_blocks,),
        in_specs=[
            pl.BlockSpec((SEQ, H), lambda i: (i, 0)),
            pl.BlockSpec((SEQ, H), lambda i: (0, 0)),
            pl.BlockSpec((1, H), lambda i: (0, 0)),
            pl.BlockSpec((1, H), lambda i: (0, 0)),
            pl.BlockSpec((1, H), lambda i: (0, 0)),
        ],
        out_specs=pl.BlockSpec((SEQ, H), lambda i: (i, 0)),
        out_shape=jax.ShapeDtypeStruct((n_rows, H), jnp.float32),
        name="bert_embed_ln_tc",
    )(gathered, position_embeddings, tte_row, gamma2d, beta2d)


def kernel(input_ids, word_embeddings, position_embeddings,
           token_type_embeddings, ln_gamma, ln_beta):
    b, seq = input_ids.shape
    n_rows = b * seq
    nparts = 1
    part = n_rows // nparts
    ids_flat = input_ids.reshape(n_rows)
    tte_row = token_type_embeddings[0].reshape(1, H)
    gamma2d = ln_gamma.reshape(1, H)
    beta2d = ln_beta.reshape(1, H)
    outs = []
    for p in range(nparts):
        gathered = _sc_gather(lax.dynamic_slice_in_dim(ids_flat, p * part, part),
                              word_embeddings, part)
        outs.append(_tc_ln(gathered, position_embeddings, tte_row,
                           gamma2d, beta2d, part))
    return jnp.concatenate(outs, axis=0).reshape(b, seq, H)
